# baseline (device time: 11759 ns/iter reference)
import jax
import jax.numpy as jnp
from jax import lax
from jax.experimental import pallas as pl
from jax.experimental.pallas import tpu as pltpu

N_DEV = 4
N_HALF = 2


def kernel(x, Wg, Wu, Wd):
    m, _ = x.shape
    mh = m // N_HALF

    def body(x_ref, wg_ref, wu_ref, wd_ref, out_ref, comm_ref,
             send_sems, recv_sems):
        my_pos = lax.axis_index("i")

        barrier_sem = pltpu.get_barrier_semaphore()
        for d in range(1, N_DEV):
            pl.semaphore_signal(
                barrier_sem, inc=1,
                device_id=((my_pos + d) % N_DEV,),
                device_id_type=pl.DeviceIdType.MESH,
            )

        partials = []
        rdmas = []
        for hf in range(N_HALF):
            xb = x_ref[pl.ds(hf * mh, mh), :]
            gate = jnp.dot(xb, wg_ref[:, :],
                           preferred_element_type=jnp.float32)
            up = jnp.dot(xb, wu_ref[:, :],
                         preferred_element_type=jnp.float32)
            hidden = gate * (up * jax.nn.sigmoid(up))
            partial = jnp.dot(hidden.astype(jnp.bfloat16), wd_ref[:, :],
                              preferred_element_type=jnp.float32)
            partials.append(partial)
            comm_ref[0, hf, :, :] = partial.astype(jnp.bfloat16)

            if hf == 0:
                pl.semaphore_wait(barrier_sem, N_DEV - 1)

            for d in range(1, N_DEV):
                rdma = pltpu.make_async_remote_copy(
                    src_ref=comm_ref.at[0, hf],
                    dst_ref=comm_ref.at[d, hf],
                    send_sem=send_sems.at[d - 1, hf],
                    recv_sem=recv_sems.at[d - 1, hf],
                    device_id=((my_pos + d) % N_DEV,),
                    device_id_type=pl.DeviceIdType.MESH,
                )
                rdma.start()
                rdmas.append(rdma)

        for hf in range(N_HALF):
            for d in range(1, N_DEV):
                rdmas[hf * (N_DEV - 1) + (d - 1)].wait_recv()
            acc = partials[hf]
            for d in range(1, N_DEV):
                acc = acc + comm_ref[d, hf, :, :].astype(jnp.float32)
            out_ref[pl.ds(hf * mh, mh), :] = acc.astype(jnp.bfloat16)

        for rdma in rdmas:
            rdma.wait_send()

    return pl.pallas_call(
        body,
        out_shape=jax.ShapeDtypeStruct((m, m), jnp.bfloat16),
        in_specs=[pl.BlockSpec(memory_space=pltpu.VMEM)] * 4,
        out_specs=pl.BlockSpec(memory_space=pltpu.VMEM),
        scratch_shapes=[
            pltpu.VMEM((N_DEV, N_HALF, mh, m), jnp.bfloat16),
            pltpu.SemaphoreType.DMA((N_DEV - 1, N_HALF)),
            pltpu.SemaphoreType.DMA((N_DEV - 1, N_HALF)),
        ],
        compiler_params=pltpu.CompilerParams(collective_id=0),
    )(
        x.astype(jnp.bfloat16),
        Wg.astype(jnp.bfloat16),
        Wu.astype(jnp.bfloat16),
        Wd.astype(jnp.bfloat16),
    )
